# Initial kernel scaffold; baseline (speedup 1.0000x reference)
import jax
import jax.numpy as jnp
from jax.experimental import pallas as pl


def kernel(center_word, context_word, neg_samples, center_table, context_table):
    raise NotImplementedError("write your pallas kernel here")


# speedup vs baseline: 1.5238x; 1.5238x over previous
import functools

import jax
import jax.numpy as jnp
from jax import lax
from jax.experimental import pallas as pl
from jax.experimental.pallas import tpu as pltpu
from jax.experimental.pallas import tpu_sc as plsc


def _make_sc_dots(B, NEG, D, V):
    info = plsc.get_sparse_core_info()
    NC, NS, L = info.num_cores, info.num_subcores, info.num_lanes
    NW = NC * NS
    bpw = B // NW
    G = bpw // L

    mesh = plsc.VectorSubcoreMesh(core_axis_name="c", subcore_axis_name="s")

    @functools.partial(
        pl.kernel,
        mesh=mesh,
        compiler_params=pltpu.CompilerParams(use_tc_tiling_on_sc=False),
        out_type=(
            jax.ShapeDtypeStruct((B, L), jnp.float32),
            jax.ShapeDtypeStruct((NEG, B, L), jnp.float32),
        ),
        scratch_types=[
            pltpu.VMEM((bpw,), jnp.int32),
            pltpu.VMEM((NEG, bpw), jnp.int32),
            pltpu.VMEM((bpw, D), jnp.float32),
            pltpu.VMEM((bpw, D), jnp.float32),
            pltpu.VMEM((bpw, D), jnp.float32),
            pltpu.VMEM((bpw, L), jnp.float32),
            pltpu.VMEM((NEG, bpw, L), jnp.float32),
            pltpu.SemaphoreType.DMA,
        ],
    )
    def sc_dots(cw, ctw, negT, ctab, xtab, corr_out, negd_out,
                idx_v, idxn_v, rows_c, rows_x, rows_n, corr_v, negd_v, sem):
        wid = lax.axis_index("s") * NC + lax.axis_index("c")
        base = wid * bpw

        pltpu.sync_copy(cw.at[pl.ds(base, bpw)], idx_v)
        pltpu.async_copy(ctab.at[idx_v], rows_c, sem).wait()
        pltpu.sync_copy(ctw.at[pl.ds(base, bpw)], idx_v)
        pltpu.async_copy(xtab.at[idx_v], rows_x, sem).wait()
        pltpu.sync_copy(negT.at[:, pl.ds(base, bpw)], idxn_v)

        def sample_partial(ra, rb, i):
            acc = ra[i, pl.ds(0, L)] * rb[i, pl.ds(0, L)]
            for j in range(1, D // L):
                acc = acc + ra[i, pl.ds(j * L, L)] * rb[i, pl.ds(j * L, L)]
            return acc

        def corr_body(g, carry):
            for j in range(L):
                i = g * L + j
                corr_v[i, :] = sample_partial(rows_c, rows_x, i)
            return carry

        lax.fori_loop(0, G, corr_body, 0)

        def neg_body(k, carry):
            pltpu.async_copy(xtab.at[idxn_v.at[k]], rows_n, sem).wait()

            def gbody(g, c2):
                for j in range(L):
                    i = g * L + j
                    negd_v[k, i, :] = sample_partial(rows_n, rows_c, i)
                return c2

            lax.fori_loop(0, G, gbody, 0)
            return carry

        lax.fori_loop(0, NEG, neg_body, 0)

        pltpu.sync_copy(corr_v, corr_out.at[pl.ds(base, bpw), :])
        pltpu.sync_copy(negd_v, negd_out.at[:, pl.ds(base, bpw), :])

    return sc_dots


def _logsig(x):
    return jnp.minimum(x, 0.0) - jnp.log1p(jnp.exp(-jnp.abs(x)))


def _make_tc_broadcast(B, NEG, L, TIL=256):
    grid = B // TIL

    def body(corr_ref, negd_ref, out_ref, a_s, b_s):
        i = pl.program_id(0)

        @pl.when(i == 0)
        def _():
            corr = jnp.sum(corr_ref[:, :], axis=1, keepdims=True)
            a_s[:, :] = _logsig(corr)
            negd = jnp.sum(negd_ref[:, :, :], axis=2)
            b_s[:, :] = jnp.sum(_logsig(negd), axis=0, keepdims=True)

        a_blk = a_s[pl.ds(i * TIL, TIL), :]
        out_ref[:, :] = -(a_blk + b_s[:, :])

    return pl.pallas_call(
        body,
        grid=(grid,),
        in_specs=[
            pl.BlockSpec((B, L), lambda i: (0, 0)),
            pl.BlockSpec((NEG, B, L), lambda i: (0, 0, 0)),
        ],
        out_specs=pl.BlockSpec((TIL, B), lambda i: (i, 0)),
        out_shape=jax.ShapeDtypeStruct((B, B), jnp.float32),
        scratch_shapes=[
            pltpu.VMEM((B, 1), jnp.float32),
            pltpu.VMEM((1, B), jnp.float32),
        ],
    )


def kernel(center_word, context_word, neg_samples, center_table, context_table):
    B = center_word.shape[0]
    NEG = neg_samples.shape[1]
    V, D = center_table.shape
    L = 16

    neg_t = neg_samples.T.astype(jnp.int32)
    cw = center_word.astype(jnp.int32)
    ctw = context_word.astype(jnp.int32)

    sc = _make_sc_dots(B, NEG, D, V)
    corr_p, negd_p = sc(cw, ctw, neg_t, center_table, context_table)

    tc = _make_tc_broadcast(B, NEG, L)
    out = tc(corr_p, negd_p)
    return out[:, :, None]


# speedup vs baseline: 1.7202x; 1.1289x over previous
import functools

import jax
import jax.numpy as jnp
from jax import lax
from jax.experimental import pallas as pl
from jax.experimental.pallas import tpu as pltpu
from jax.experimental.pallas import tpu_sc as plsc


def _make_sc_dots(B, NEG, D, V):
    info = plsc.get_sparse_core_info()
    NC, NS, L = info.num_cores, info.num_subcores, info.num_lanes
    NW = NC * NS
    bpw = B // NW
    RW = bpw * L // 128

    mesh = plsc.VectorSubcoreMesh(core_axis_name="c", subcore_axis_name="s")

    @functools.partial(
        pl.kernel,
        mesh=mesh,
        out_type=(
            jax.ShapeDtypeStruct((B, L), jnp.float32),
            jax.ShapeDtypeStruct((NEG, B * L // 128, 128), jnp.float32),
        ),
        scratch_types=[
            pltpu.VMEM((bpw,), jnp.int32),
            pltpu.VMEM((bpw,), jnp.int32),
            pltpu.VMEM((bpw,), jnp.int32),
            pltpu.VMEM((NEG, bpw), jnp.int32),
            pltpu.VMEM((NEG, bpw), jnp.int32),
            pltpu.VMEM((bpw, 128), jnp.float32),
            pltpu.VMEM((bpw, 128), jnp.float32),
            pltpu.VMEM((bpw, 128), jnp.float32),
            pltpu.VMEM((bpw, L), jnp.float32),
            pltpu.VMEM((NEG, RW, 128), jnp.float32),
            pltpu.SemaphoreType.DMA,
        ],
    )
    def sc_dots(cwh, cwo, ctwh, ctwo, negh, nego, ctab2, xtab2,
                corr_out, negd_out,
                idx_v, off_c, off_o, idxn_v, offn_v,
                rows_c, rows_x, rows_n, corr_v, negd_v, sem):
        wid = lax.axis_index("s") * NC + lax.axis_index("c")
        base = wid * bpw

        pltpu.sync_copy(cwh.at[pl.ds(base, bpw)], idx_v)
        pltpu.async_copy(ctab2.at[idx_v], rows_c, sem).wait()
        pltpu.sync_copy(ctwh.at[pl.ds(base, bpw)], idx_v)
        pltpu.async_copy(xtab2.at[idx_v], rows_x, sem).wait()
        pltpu.sync_copy(cwo.at[pl.ds(base, bpw)], off_c)
        pltpu.sync_copy(ctwo.at[pl.ds(base, bpw)], off_o)
        pltpu.sync_copy(negh.at[:, pl.ds(base, bpw)], idxn_v)
        pltpu.sync_copy(nego.at[:, pl.ds(base, bpw)], offn_v)

        def sample_partial(ra, oa, rb, ob, i):
            acc = None
            for j in range(D // L):
                pa = ra[i, pl.ds(oa + j * L, L)] * rb[i, pl.ds(ob + j * L, L)]
                acc = pa if acc is None else acc + pa
            return acc

        def corr_body(g, carry):
            ovc = off_c[pl.ds(g * L, L)]
            ovx = off_o[pl.ds(g * L, L)]
            for u in range(L):
                i = g * L + u
                p = sample_partial(rows_c, ovc[u], rows_x, ovx[u], i)
                corr_v[i, :] = p
            return carry

        lax.fori_loop(0, bpw // L, corr_body, 0)

        def neg_body(k, carry):
            pltpu.async_copy(xtab2.at[idxn_v.at[k]], rows_n, sem).wait()

            def gbody(g, c2):
                ovc = off_c[pl.ds(g * L, L)]
                ovn = offn_v[k, pl.ds(g * L, L)]
                for u in range(L):
                    i = g * L + u
                    p = sample_partial(rows_n, ovn[u], rows_c, ovc[u], i)
                    negd_v[k, i // 8, pl.ds((i % 8) * L, L)] = p
                return c2

            lax.fori_loop(0, bpw // L, gbody, 0)
            return carry

        lax.fori_loop(0, NEG, neg_body, 0)

        pltpu.sync_copy(corr_v, corr_out.at[pl.ds(base, bpw), :])
        pltpu.sync_copy(negd_v, negd_out.at[:, pl.ds(wid * RW, RW), :])

    return sc_dots


def _logsig(x):
    return jnp.minimum(x, 0.0) - jnp.log1p(jnp.exp(-jnp.abs(x)))


def _make_tc_broadcast(B, NEG, L, TIL=256):
    grid = B // TIL
    R = B * L // 128

    def body(corr_ref, negd_ref, out_ref, a_s, b_s):
        t = pl.program_id(0)

        @pl.when(t == 0)
        def _():
            ones_l = jnp.ones((L, 1), jnp.float32)
            cd = jnp.dot(corr_ref[:, :], ones_l,
                         preferred_element_type=jnp.float32)
            a_s[:, :] = _logsig(cd)

            fold = (lax.broadcasted_iota(jnp.int32, (128, 8), 0) // L ==
                    lax.broadcasted_iota(jnp.int32, (128, 8), 1)
                    ).astype(jnp.float32)
            nd = jnp.dot(jnp.reshape(negd_ref[:, :, :], (NEG * R, 128)), fold,
                         preferred_element_type=jnp.float32)
            nl = _logsig(nd)
            b8 = nl[0:R, :]
            for k in range(1, NEG):
                b8 = b8 + nl[k * R:(k + 1) * R, :]
            exp8 = (lax.broadcasted_iota(jnp.int32, (8, B), 1) % 8 ==
                    lax.broadcasted_iota(jnp.int32, (8, B), 0)
                    ).astype(jnp.float32)
            bex = jnp.dot(b8, exp8,
                          preferred_element_type=jnp.float32)
            rmask = (lax.broadcasted_iota(jnp.int32, (R, B), 1) // 8 ==
                     lax.broadcasted_iota(jnp.int32, (R, B), 0)
                     ).astype(jnp.float32)
            b_s[:, :] = jnp.dot(jnp.ones((1, R), jnp.float32), bex * rmask,
                                preferred_element_type=jnp.float32)

        a_blk = a_s[pl.ds(t * TIL, TIL), :]
        out_ref[:, :] = -(a_blk + b_s[:, :])

    return pl.pallas_call(
        body,
        grid=(grid,),
        in_specs=[
            pl.BlockSpec((B, L), lambda i: (0, 0)),
            pl.BlockSpec((NEG, R, 128), lambda i: (0, 0, 0)),
        ],
        out_specs=pl.BlockSpec((TIL, B), lambda i: (i, 0)),
        out_shape=jax.ShapeDtypeStruct((B, B), jnp.float32),
        scratch_shapes=[
            pltpu.VMEM((B, 1), jnp.float32),
            pltpu.VMEM((1, B), jnp.float32),
        ],
    )


def kernel(center_word, context_word, neg_samples, center_table, context_table):
    B = center_word.shape[0]
    NEG = neg_samples.shape[1]
    V, D = center_table.shape
    L = 16

    ctab2 = center_table.reshape(V // 2, 2 * D)
    xtab2 = context_table.reshape(V // 2, 2 * D)

    cw = center_word.astype(jnp.int32)
    ctw = context_word.astype(jnp.int32)
    neg_t = neg_samples.T.astype(jnp.int32)

    cwh, cwo = cw >> 1, (cw & 1) * D
    ctwh, ctwo = ctw >> 1, (ctw & 1) * D
    negh, nego = neg_t >> 1, (neg_t & 1) * D

    sc = _make_sc_dots(B, NEG, D, V)
    corr_p, negd_p = sc(cwh, cwo, ctwh, ctwo, negh, nego, ctab2, xtab2)

    tc = _make_tc_broadcast(B, NEG, L)
    out = tc(corr_p, negd_p)
    return out[:, :, None]


# speedup vs baseline: 2.0950x; 1.2179x over previous
import functools

import jax
import jax.numpy as jnp
from jax import lax
from jax.experimental import pallas as pl
from jax.experimental.pallas import tpu as pltpu
from jax.experimental.pallas import tpu_sc as plsc


def _make_sc_dots(B, NEG, D, V):
    info = plsc.get_sparse_core_info()
    NC, NS, L = info.num_cores, info.num_subcores, info.num_lanes
    NW = NC * NS
    bpw = B // NW
    RW = bpw * L // 128

    mesh = plsc.VectorSubcoreMesh(core_axis_name="c", subcore_axis_name="s")

    @functools.partial(
        pl.kernel,
        mesh=mesh,
        out_type=(
            jax.ShapeDtypeStruct((B, L), jnp.float32),
            jax.ShapeDtypeStruct((NEG, B * L // 128, 128), jnp.float32),
        ),
        scratch_types=[
            pltpu.VMEM((bpw,), jnp.int32),
            pltpu.VMEM((bpw,), jnp.int32),
            pltpu.VMEM((bpw,), jnp.int32),
            pltpu.VMEM((NEG, bpw), jnp.int32),
            pltpu.VMEM((NEG, bpw), jnp.int32),
            pltpu.VMEM((bpw, 128), jnp.float32),
            pltpu.VMEM((bpw, 128), jnp.float32),
            pltpu.VMEM((bpw, 128), jnp.float32),
            pltpu.VMEM((bpw, L), jnp.float32),
            pltpu.VMEM((NEG, RW, 128), jnp.float32),
            pltpu.SemaphoreType.DMA,
        ],
    )
    def sc_dots(cwh, cwo, ctwh, ctwo, negh, nego, ctab2, xtab2,
                corr_out, negd_out,
                idx_v, off_c, off_o, idxn_v, offn_v,
                rows_c, rows_x, rows_n, corr_v, negd_v, sem):
        wid = lax.axis_index("s") * NC + lax.axis_index("c")
        base = wid * bpw

        pltpu.sync_copy(cwh.at[pl.ds(base, bpw)], idx_v)
        pltpu.async_copy(ctab2.at[idx_v], rows_c, sem).wait()
        pltpu.sync_copy(ctwh.at[pl.ds(base, bpw)], idx_v)
        pltpu.async_copy(xtab2.at[idx_v], rows_x, sem).wait()
        pltpu.sync_copy(cwo.at[pl.ds(base, bpw)], off_c)
        pltpu.sync_copy(ctwo.at[pl.ds(base, bpw)], off_o)
        pltpu.sync_copy(negh.at[:, pl.ds(base, bpw)], idxn_v)
        pltpu.sync_copy(nego.at[:, pl.ds(base, bpw)], offn_v)

        def sample_partial(ra, oa, rb, ob, i):
            acc = None
            for j in range(D // L):
                pa = ra[i, pl.ds(oa + j * L, L)] * rb[i, pl.ds(ob + j * L, L)]
                acc = pa if acc is None else acc + pa
            return acc

        def corr_body(g, carry):
            ovc = off_c[pl.ds(g * L, L)]
            ovx = off_o[pl.ds(g * L, L)]
            for u in range(L):
                i = g * L + u
                p = sample_partial(rows_c, ovc[u], rows_x, ovx[u], i)
                corr_v[i, :] = p
            return carry

        lax.fori_loop(0, bpw // L, corr_body, 0)

        def neg_body(k, carry):
            pltpu.async_copy(xtab2.at[idxn_v.at[k]], rows_n, sem).wait()

            def gbody(g, c2):
                ovc = off_c[pl.ds(g * L, L)]
                ovn = offn_v[k, pl.ds(g * L, L)]
                for u in range(L):
                    i = g * L + u
                    p = sample_partial(rows_n, ovn[u], rows_c, ovc[u], i)
                    negd_v[k, i // 8, pl.ds((i % 8) * L, L)] = p
                return c2

            lax.fori_loop(0, bpw // L, gbody, 0)
            return carry

        lax.fori_loop(0, NEG, neg_body, 0)

        pltpu.sync_copy(corr_v, corr_out.at[pl.ds(base, bpw), :])
        pltpu.sync_copy(negd_v, negd_out.at[:, pl.ds(wid * RW, RW), :])

    return sc_dots


def _logsig(x):
    return jnp.minimum(x, 0.0) - jnp.log1p(jnp.exp(-jnp.abs(x)))


def _make_tc_broadcast(B, NEG, L, TIL=256):
    grid = B // TIL
    R = B * L // 128

    def body(corr_ref, negd_ref, out_ref, a_s, b_s):
        t = pl.program_id(0)

        @pl.when(t == 0)
        def _():
            ones_l = jnp.ones((L, 1), jnp.float32)
            cd = jnp.dot(corr_ref[:, :], ones_l,
                         preferred_element_type=jnp.float32)
            a_s[:, :] = _logsig(cd)

            fold = (lax.broadcasted_iota(jnp.int32, (128, 8), 0) // L ==
                    lax.broadcasted_iota(jnp.int32, (128, 8), 1)
                    ).astype(jnp.float32)
            nd = jnp.dot(jnp.reshape(negd_ref[:, :, :], (NEG * R, 128)), fold,
                         preferred_element_type=jnp.float32)
            nl = _logsig(nd)
            b8 = nl[0:R, :]
            for k in range(1, NEG):
                b8 = b8 + nl[k * R:(k + 1) * R, :]
            e8 = (lax.broadcasted_iota(jnp.int32, (8, 128), 1) % 8 ==
                  lax.broadcasted_iota(jnp.int32, (8, 128), 0)
                  ).astype(jnp.float32)
            bex = jnp.dot(b8, e8,
                          preferred_element_type=jnp.float32)
            lmask = (lax.broadcasted_iota(jnp.int32, (R, 128), 1) // 8 ==
                     lax.broadcasted_iota(jnp.int32, (R, 128), 0) % L
                     ).astype(jnp.float32)
            mq = (lax.broadcasted_iota(jnp.int32, (32, R), 1) // L ==
                  lax.broadcasted_iota(jnp.int32, (32, R), 0)
                  ).astype(jnp.float32)
            b32 = jnp.dot(mq, bex * lmask,
                          preferred_element_type=jnp.float32)
            b_s[:, :, :] = b32[None, :, :]

        a_blk = a_s[pl.ds(t * TIL, TIL), :]
        out_ref[:, :, :] = -(a_blk[:, :, None] + b_s[:, :, :])

    return pl.pallas_call(
        body,
        grid=(grid,),
        in_specs=[
            pl.BlockSpec((B, L), lambda i: (0, 0)),
            pl.BlockSpec((NEG, R, 128), lambda i: (0, 0, 0)),
        ],
        out_specs=pl.BlockSpec((TIL, 32, 128), lambda i: (i, 0, 0)),
        out_shape=jax.ShapeDtypeStruct((B, 32, 128), jnp.float32),
        scratch_shapes=[
            pltpu.VMEM((B, 1), jnp.float32),
            pltpu.VMEM((1, 32, 128), jnp.float32),
        ],
    )


def kernel(center_word, context_word, neg_samples, center_table, context_table):
    B = center_word.shape[0]
    NEG = neg_samples.shape[1]
    V, D = center_table.shape
    L = 16

    ctab2 = center_table.reshape(V // 2, 2 * D)
    xtab2 = context_table.reshape(V // 2, 2 * D)

    cw = center_word.astype(jnp.int32)
    ctw = context_word.astype(jnp.int32)
    neg_t = neg_samples.T.astype(jnp.int32)

    cwh, cwo = cw >> 1, (cw & 1) * D
    ctwh, ctwo = ctw >> 1, (ctw & 1) * D
    negh, nego = neg_t >> 1, (neg_t & 1) * D

    sc = _make_sc_dots(B, NEG, D, V)
    corr_p, negd_p = sc(cwh, cwo, ctwh, ctwo, negh, nego, ctab2, xtab2)

    tc = _make_tc_broadcast(B, NEG, L)
    out = tc(corr_p, negd_p)
    return jnp.reshape(out, (B, B, 1))


# speedup vs baseline: 2.6689x; 1.2739x over previous
import functools

import jax
import jax.numpy as jnp
from jax import lax
from jax.experimental import pallas as pl
from jax.experimental.pallas import tpu as pltpu
from jax.experimental.pallas import tpu_sc as plsc


def _make_sc_dots(B, NEG, D, V):
    info = plsc.get_sparse_core_info()
    NC, NS, L = info.num_cores, info.num_subcores, info.num_lanes
    NW = NC * NS
    bpw = B // NW
    RW = bpw * L // 128

    mesh = plsc.VectorSubcoreMesh(core_axis_name="c", subcore_axis_name="s")

    @functools.partial(
        pl.kernel,
        mesh=mesh,
        out_type=(
            jax.ShapeDtypeStruct((B, L), jnp.float32),
            jax.ShapeDtypeStruct((NEG, B * L // 128, 128), jnp.float32),
        ),
        scratch_types=[
            pltpu.VMEM((bpw,), jnp.int32),
            pltpu.VMEM((bpw,), jnp.int32),
            pltpu.VMEM((NEG, bpw), jnp.int32),
            pltpu.VMEM((bpw, 128), jnp.float32),
            pltpu.VMEM((bpw, 128), jnp.float32),
            pltpu.VMEM((bpw, 128), jnp.float32),
            pltpu.VMEM((bpw, 128), jnp.float32),
            pltpu.VMEM((bpw, L), jnp.float32),
            pltpu.VMEM((NEG, RW, 128), jnp.float32),
            pltpu.SemaphoreType.DMA,
            pltpu.SemaphoreType.DMA,
            pltpu.SemaphoreType.DMA,
            pltpu.SemaphoreType.DMA,
        ],
    )
    def sc_dots(cw, ctw, negT, ctab, xtab, corr_out, negd_out,
                idx_v, idx_v2, idxn_v, rows_c, rows_x, rows_n0, rows_n1,
                corr_v, negd_v, semA, semB, sem0, sem1):
        wid = lax.axis_index("s") * NC + lax.axis_index("c")
        base = wid * bpw

        pltpu.sync_copy(cw.at[pl.ds(base, bpw)], idx_v)
        cpc = pltpu.async_copy(ctab.at[idx_v], rows_c, semA)
        pltpu.sync_copy(ctw.at[pl.ds(base, bpw)], idx_v2)
        cpx = pltpu.async_copy(xtab.at[idx_v2], rows_x, semB)
        pltpu.sync_copy(negT.at[:, pl.ds(base, bpw)], idxn_v)

        nbufs = (rows_n0, rows_n1)
        nsems = (sem0, sem1)
        pltpu.async_copy(xtab.at[idxn_v.at[0]], rows_n0, sem0)
        pltpu.async_copy(xtab.at[idxn_v.at[1]], rows_n1, sem1)

        def sample_partial(ra, rb, i):
            acc = None
            for j in range(D // L):
                pa = ra[i, pl.ds(j * L, L)] * rb[i, pl.ds(j * L, L)]
                acc = pa if acc is None else acc + pa
            return acc

        cpc.wait()
        cpx.wait()

        def corr_body(g, carry):
            for u in range(L):
                i = g * L + u
                corr_v[i, :] = sample_partial(rows_c, rows_x, i)
            return carry

        lax.fori_loop(0, bpw // L, corr_body, 0)

        def kk_body(kk, carry):
            for b in range(2):
                k = 2 * kk + b
                buf = nbufs[b]
                pltpu.make_async_copy(xtab.at[idxn_v.at[0]], buf,
                                      nsems[b]).wait()

                def gbody(g, c2, k=k, buf=buf):
                    for u in range(L):
                        i = g * L + u
                        p = sample_partial(buf, rows_c, i)
                        negd_v[k, i // 8, pl.ds((i % 8) * L, L)] = p
                    return c2

                lax.fori_loop(0, bpw // L, gbody, 0)

                @pl.when(k + 2 < NEG)
                def _(k=k, b=b, buf=buf):
                    pltpu.async_copy(xtab.at[idxn_v.at[k + 2]], buf, nsems[b])

            return carry

        lax.fori_loop(0, NEG // 2, kk_body, 0)

        pltpu.sync_copy(corr_v, corr_out.at[pl.ds(base, bpw), :])
        pltpu.sync_copy(negd_v, negd_out.at[:, pl.ds(wid * RW, RW), :])

    return sc_dots


def _logsig(x):
    return jnp.minimum(x, 0.0) - jnp.log1p(jnp.exp(-jnp.abs(x)))


def _make_tc_broadcast(B, NEG, L, TIL=256):
    grid = B // TIL
    R = B * L // 128

    def body(corr_ref, negd_ref, out_ref, a_s, b_s):
        t = pl.program_id(0)

        @pl.when(t == 0)
        def _():
            ones_l = jnp.ones((L, 1), jnp.float32)
            cd = jnp.dot(corr_ref[:, :], ones_l,
                         preferred_element_type=jnp.float32)
            a_s[:, :] = _logsig(cd)

            fold = (lax.broadcasted_iota(jnp.int32, (128, 8), 0) // L ==
                    lax.broadcasted_iota(jnp.int32, (128, 8), 1)
                    ).astype(jnp.float32)
            nd = jnp.dot(jnp.reshape(negd_ref[:, :, :], (NEG * R, 128)), fold,
                         preferred_element_type=jnp.float32)
            nl = _logsig(nd)
            b8 = nl[0:R, :]
            for k in range(1, NEG):
                b8 = b8 + nl[k * R:(k + 1) * R, :]
            e8 = (lax.broadcasted_iota(jnp.int32, (8, 128), 1) % 8 ==
                  lax.broadcasted_iota(jnp.int32, (8, 128), 0)
                  ).astype(jnp.float32)
            bex = jnp.dot(b8, e8,
                          preferred_element_type=jnp.float32)
            lmask = (lax.broadcasted_iota(jnp.int32, (R, 128), 1) // 8 ==
                     lax.broadcasted_iota(jnp.int32, (R, 128), 0) % L
                     ).astype(jnp.float32)
            mq = (lax.broadcasted_iota(jnp.int32, (32, R), 1) // L ==
                  lax.broadcasted_iota(jnp.int32, (32, R), 0)
                  ).astype(jnp.float32)
            b32 = jnp.dot(mq, bex * lmask,
                          preferred_element_type=jnp.float32)
            b_s[:, :, :] = b32[None, :, :]

        a_blk = a_s[pl.ds(t * TIL, TIL), :]
        out_ref[:, :, :] = -(a_blk[:, :, None] + b_s[:, :, :])

    return pl.pallas_call(
        body,
        grid=(grid,),
        in_specs=[
            pl.BlockSpec((B, L), lambda i: (0, 0)),
            pl.BlockSpec((NEG, R, 128), lambda i: (0, 0, 0)),
        ],
        out_specs=pl.BlockSpec((TIL, 32, 128), lambda i: (i, 0, 0)),
        out_shape=jax.ShapeDtypeStruct((B, 32, 128), jnp.float32),
        scratch_shapes=[
            pltpu.VMEM((B, 1), jnp.float32),
            pltpu.VMEM((1, 32, 128), jnp.float32),
        ],
    )


def kernel(center_word, context_word, neg_samples, center_table, context_table):
    B = center_word.shape[0]
    NEG = neg_samples.shape[1]
    V, D = center_table.shape
    L = 16

    ctabp = jnp.pad(center_table, ((0, 0), (0, 128 - D)))
    xtabp = jnp.pad(context_table, ((0, 0), (0, 128 - D)))

    cw = center_word.astype(jnp.int32)
    ctw = context_word.astype(jnp.int32)
    neg_t = neg_samples.T.astype(jnp.int32)

    sc = _make_sc_dots(B, NEG, D, V)
    corr_p, negd_p = sc(cw, ctw, neg_t, ctabp, xtabp)

    tc = _make_tc_broadcast(B, NEG, L)
    out = tc(corr_p, negd_p)
    return jnp.reshape(out, (B, B, 1))
